# baseline (speedup 1.0000x reference)
import functools

import jax
import jax.numpy as jnp
from jax import lax
from jax.experimental import pallas as pl
from jax.experimental.pallas import tpu as pltpu
from jax.experimental.pallas import tpu_sc as plsc

N = 10000
E = 320000
D = 128

NC = 2
NS = 16
NW = NC * NS

NPAD = 10240
RPT = NPAD // NS

EB = 80
EPT = E // NW
ENB = EPT // EB

HPT = E // NS
HNB = HPT // EB

_f32 = jnp.float32
_i32 = jnp.int32

_sc_mesh = plsc.VectorSubcoreMesh(
    core_axis_name="c", subcore_axis_name="s", num_cores=NC, num_subcores=NS)


HK = 5
HRND = HNB // HK


@functools.partial(
    pl.kernel,
    out_type=jax.ShapeDtypeStruct((6 * NPAD,), _f32),
    mesh=_sc_mesh,
    scratch_types=[
        pltpu.VMEM((HK, EB), _i32),
        pltpu.VMEM((EB,), _f32),
        pltpu.VMEM((RPT,), _f32),
        pltpu.VMEM_SHARED((NPAD,), _f32),
        pltpu.VMEM_SHARED((NPAD,), _f32),
        pltpu.VMEM_SHARED((NPAD,), _f32),
    ] + [pltpu.SemaphoreType.DMA] * (2 * HK),
)
def _deg_kernel(s0, s1, s2, d0, d1, d2, ones_hbm, zeros_hbm, deg_hbm,
                idx_v, ones_v, z_v, h0, h1, h2, *sems):
    c = lax.axis_index("c")
    s = lax.axis_index("s")
    hists = [h0, h1, h2]
    srcs = [s0, s1, s2]
    dsts = [d0, d1, d2]
    ld_sems = sems[:HK]
    sc_sems = sems[HK:]

    pltpu.sync_copy(ones_hbm, ones_v)
    pltpu.sync_copy(zeros_hbm, z_v)
    for h in range(3):
        pltpu.sync_copy(z_v, hists[h].at[pl.ds(s * RPT, RPT)])
    plsc.subcore_barrier()

    for h in range(3):
        def _fire_load(k, i):
            off = s * HPT + i * EB

            @pl.when(c == 0)
            def _():
                pltpu.async_copy(srcs[h].at[pl.ds(off, EB)],
                                 idx_v.at[k], ld_sems[k])

            @pl.when(c == 1)
            def _():
                pltpu.async_copy(dsts[h].at[pl.ds(off, EB)],
                                 idx_v.at[k], ld_sems[k])

        def _wait_load(k):
            pltpu.make_async_copy(srcs[h].at[pl.ds(0, EB)],
                                  idx_v.at[k], ld_sems[k]).wait()

        def _fire_scatter(k):
            pltpu.make_async_copy(
                ones_v, hists[h].at[idx_v.at[k]],
                sc_sems[k]).start(add=True)

        def _drain_scatter(k):
            pltpu.make_async_copy(
                ones_v, hists[h].at[idx_v.at[k]], sc_sems[k]).wait()

        for k in range(HK):
            _fire_load(k, k)

        @pl.loop(0, HRND - 1)
        def _round(o):
            for k in range(HK):
                _wait_load(k)
                _fire_scatter(k)
            for k in range(HK):
                _drain_scatter(k)
                _fire_load(k, (o + 1) * HK + k)

        for k in range(HK):
            _wait_load(k)
            _fire_scatter(k)
        for k in range(HK):
            _drain_scatter(k)

    plsc.subcore_barrier()
    for h in range(3):
        @pl.when(c == 0)
        def _():
            pltpu.sync_copy(hists[h].at[pl.ds(s * RPT, RPT)],
                            deg_hbm.at[pl.ds(h * NPAD + s * RPT, RPT)])

        @pl.when(c == 1)
        def _():
            pltpu.sync_copy(hists[h].at[pl.ds(s * RPT, RPT)],
                            deg_hbm.at[pl.ds((h + 3) * NPAD + s * RPT, RPT)])


AK = 4
AEB = 50
AENB = EPT // AEB
ARND = AENB // AK
ZR = 64

_agg_scratch = [
    pltpu.VMEM((AK, 2, AEB), _i32),
    pltpu.VMEM((AK, AEB, D), _f32),
    pltpu.VMEM((ZR, D), _f32),
    pltpu.VMEM_SHARED((NPAD, D), _f32),
] + [pltpu.SemaphoreType.DMA] * (3 * AK)


@functools.partial(
    pl.kernel,
    out_type=(jax.ShapeDtypeStruct((3, NPAD, D), _f32),
              jax.ShapeDtypeStruct((3, NPAD, D), _f32)),
    mesh=_sc_mesh,
    scratch_types=_agg_scratch,
)
def _agg_kernel(hs0, hs1, hs2, ed_hbm, zin_hbm, p0_hbm, p1_hbm,
                idx_v, rows_v, z_v, acc_sh, *sems):
    c = lax.axis_index("c")
    s = lax.axis_index("s")
    wid = c * NS + s
    hss = [hs0, hs1, hs2]
    l_sems = sems[:AK]
    g_sems = sems[AK:2 * AK]
    s_sems = sems[2 * AK:]
    r0 = s * RPT

    @pl.loop(0, ZR)
    def _zinit(i):
        for j in range(D // 16):
            z_v[i, pl.ds(j * 16, 16)] = jnp.zeros((16,), _f32)

    for r in range(3):
        hs = hss[r]
        base = r * (E // AEB) + wid * AENB

        def _fire_load(k, i):
            pltpu.async_copy(ed_hbm.at[base + i], idx_v.at[k], l_sems[k])

        def _wait_load(k):
            pltpu.make_async_copy(ed_hbm.at[base], idx_v.at[k],
                                  l_sems[k]).wait()

        def _fire_gather(k):
            pltpu.async_copy(hs.at[idx_v.at[k, 0]], rows_v.at[k],
                             g_sems[k])

        def _wait_gather(k):
            pltpu.make_async_copy(hs.at[idx_v.at[k, 0]], rows_v.at[k],
                                  g_sems[k]).wait()

        def _fire_scatter(k):
            pltpu.make_async_copy(rows_v.at[k], acc_sh.at[idx_v.at[k, 1]],
                                  s_sems[k]).start(add=True)

        def _drain_scatter(k):
            pltpu.make_async_copy(rows_v.at[k], acc_sh.at[idx_v.at[k, 1]],
                                  s_sems[k]).wait()

        for k in range(AK):
            _fire_load(k, k)

        for t in range(RPT // ZR):
            pltpu.sync_copy(z_v, acc_sh.at[pl.ds(r0 + t * ZR, ZR)])
        plsc.subcore_barrier()

        @pl.loop(0, ARND - 1)
        def _round(o):
            for k in range(AK):
                _wait_load(k)
                _fire_gather(k)
            for k in range(AK):
                _wait_gather(k)
                _fire_scatter(k)
            for k in range(AK):
                _drain_scatter(k)
                _fire_load(k, (o + 1) * AK + k)

        for k in range(AK):
            _wait_load(k)
            _fire_gather(k)
        for k in range(AK):
            _wait_gather(k)
            _fire_scatter(k)
        for k in range(AK):
            _drain_scatter(k)
        plsc.subcore_barrier()

        @pl.when(c == 0)
        def _():
            pltpu.sync_copy(acc_sh.at[pl.ds(r0, RPT)],
                            p0_hbm.at[r, pl.ds(r0, RPT)])

        @pl.when(c == 1)
        def _():
            pltpu.sync_copy(acc_sh.at[pl.ds(r0, RPT)],
                            p1_hbm.at[r, pl.ds(r0, RPT)])


BN = 1000
NBLK = N // BN


def _scale_body(x_ref, w0_ref, w1_ref, w2_ref, deg_ref,
                h0_ref, h1_ref, h2_ref):
    xb = x_ref[...]
    outs = [h0_ref, h1_ref, h2_ref]
    ws = [w0_ref, w1_ref, w2_ref]
    for r in range(3):
        h = jnp.dot(xb, ws[r][...], preferred_element_type=_f32)
        nrm = lax.rsqrt(jnp.maximum(deg_ref[r, 0, 0, :], 1.0))
        outs[r][...] = h * nrm[:, None]


_scale_call = pl.pallas_call(
    _scale_body,
    grid=(NBLK,),
    in_specs=[
        pl.BlockSpec((BN, D), lambda b: (b, 0)),
        pl.BlockSpec((D, D), lambda b: (0, 0)),
        pl.BlockSpec((D, D), lambda b: (0, 0)),
        pl.BlockSpec((D, D), lambda b: (0, 0)),
        pl.BlockSpec((3, 1, 1, BN), lambda b: (0, b, 0, 0)),
    ],
    out_specs=[pl.BlockSpec((BN, D), lambda b: (b, 0))] * 3,
    out_shape=[jax.ShapeDtypeStruct((N, D), _f32)] * 3,
)


def _combine_body(p0_ref, p1_ref, deg_ref, out_ref):
    acc = jnp.zeros((BN, D), _f32)
    for r in range(3):
        nrm = lax.rsqrt(jnp.maximum(deg_ref[r, 0, 0, :], 1.0))
        acc = acc + (p0_ref[r] + p1_ref[r]) * nrm[:, None]
    out_ref[...] = acc


_combine_call = pl.pallas_call(
    _combine_body,
    grid=(NBLK,),
    in_specs=[
        pl.BlockSpec((3, BN, D), lambda b: (0, b, 0)),
        pl.BlockSpec((3, BN, D), lambda b: (0, b, 0)),
        pl.BlockSpec((3, 1, 1, BN), lambda b: (0, b, 0, 0)),
    ],
    out_specs=pl.BlockSpec((BN, D), lambda b: (b, 0)),
    out_shape=jax.ShapeDtypeStruct((N, D), _f32),
)


def kernel(x, edge_index_rel0, edge_index_rel1, edge_index_rel2, W0, W1, W2):
    src = jnp.stack([edge_index_rel0[0], edge_index_rel1[0],
                     edge_index_rel2[0]])
    dst = jnp.stack([edge_index_rel0[1], edge_index_rel1[1],
                     edge_index_rel2[1]])
    ed = jnp.stack([src.reshape(3, E // AEB, AEB),
                    dst.reshape(3, E // AEB, AEB)],
                   axis=2).reshape(3 * (E // AEB), 2, AEB)

    ones_in = jnp.ones((EB,), _f32)
    zeros_in = jnp.zeros((RPT,), _f32)
    deg = _deg_kernel(src[0], src[1], src[2],
                      dst[0], dst[1], dst[2],
                      ones_in, zeros_in).reshape(6, NPAD)
    deg_out = deg[:3, :N].reshape(3, NBLK, 1, BN)
    deg_in = deg[3:, :N].reshape(3, NBLK, 1, BN)

    h0, h1, h2 = _scale_call(x, W0, W1, W2, deg_out)
    zin = jnp.zeros((RPT, D), _f32)
    p0, p1 = _agg_kernel(h0, h1, h2, ed, zin)
    out = _combine_call(p0, p1, deg_in)
    return out


# speedup vs baseline: 1.0411x; 1.0411x over previous
import functools

import jax
import jax.numpy as jnp
from jax import lax
from jax.experimental import pallas as pl
from jax.experimental.pallas import tpu as pltpu
from jax.experimental.pallas import tpu_sc as plsc

N = 10000
E = 320000
D = 128

NC = 2
NS = 16
NW = NC * NS

NPAD = 10240
RPT = NPAD // NS

EB = 80
EPT = E // NW
ENB = EPT // EB

HPT = E // NS
HNB = HPT // EB

_f32 = jnp.float32
_i32 = jnp.int32

_sc_mesh = plsc.VectorSubcoreMesh(
    core_axis_name="c", subcore_axis_name="s", num_cores=NC, num_subcores=NS)


HK = 10
HRND = HNB // HK


@functools.partial(
    pl.kernel,
    out_type=jax.ShapeDtypeStruct((6 * NPAD,), _f32),
    mesh=_sc_mesh,
    scratch_types=[
        pltpu.VMEM((HK, EB), _i32),
        pltpu.VMEM((EB,), _f32),
        pltpu.VMEM((RPT,), _f32),
        pltpu.VMEM_SHARED((NPAD,), _f32),
        pltpu.VMEM_SHARED((NPAD,), _f32),
        pltpu.VMEM_SHARED((NPAD,), _f32),
    ] + [pltpu.SemaphoreType.DMA] * (2 * HK),
)
def _deg_kernel(s0, s1, s2, d0, d1, d2, ones_hbm, zeros_hbm, deg_hbm,
                idx_v, ones_v, z_v, h0, h1, h2, *sems):
    c = lax.axis_index("c")
    s = lax.axis_index("s")
    hists = [h0, h1, h2]
    srcs = [s0, s1, s2]
    dsts = [d0, d1, d2]
    ld_sems = sems[:HK]
    sc_sems = sems[HK:]

    pltpu.sync_copy(ones_hbm, ones_v)
    pltpu.sync_copy(zeros_hbm, z_v)
    for h in range(3):
        pltpu.sync_copy(z_v, hists[h].at[pl.ds(s * RPT, RPT)])
    plsc.subcore_barrier()

    for h in range(3):
        def _fire_load(k, i):
            off = s * HPT + i * EB

            @pl.when(c == 0)
            def _():
                pltpu.async_copy(srcs[h].at[pl.ds(off, EB)],
                                 idx_v.at[k], ld_sems[k])

            @pl.when(c == 1)
            def _():
                pltpu.async_copy(dsts[h].at[pl.ds(off, EB)],
                                 idx_v.at[k], ld_sems[k])

        def _wait_load(k):
            pltpu.make_async_copy(srcs[h].at[pl.ds(0, EB)],
                                  idx_v.at[k], ld_sems[k]).wait()

        def _fire_scatter(k):
            pltpu.make_async_copy(
                ones_v, hists[h].at[idx_v.at[k]],
                sc_sems[k]).start(add=True)

        def _drain_scatter(k):
            pltpu.make_async_copy(
                ones_v, hists[h].at[idx_v.at[k]], sc_sems[k]).wait()

        for k in range(HK):
            _fire_load(k, k)

        @pl.loop(0, HRND - 1)
        def _round(o):
            for k in range(HK):
                _wait_load(k)
                _fire_scatter(k)
            for k in range(HK):
                _drain_scatter(k)
                _fire_load(k, (o + 1) * HK + k)

        for k in range(HK):
            _wait_load(k)
            _fire_scatter(k)
        for k in range(HK):
            _drain_scatter(k)

    plsc.subcore_barrier()
    for h in range(3):
        @pl.when(c == 0)
        def _():
            pltpu.sync_copy(hists[h].at[pl.ds(s * RPT, RPT)],
                            deg_hbm.at[pl.ds(h * NPAD + s * RPT, RPT)])

        @pl.when(c == 1)
        def _():
            pltpu.sync_copy(hists[h].at[pl.ds(s * RPT, RPT)],
                            deg_hbm.at[pl.ds((h + 3) * NPAD + s * RPT, RPT)])


AK = 4
AEB = 50
AENB = EPT // AEB
ARND = AENB // AK
ZR = 64

_agg_scratch = [
    pltpu.VMEM((AK, 2, AEB), _i32),
    pltpu.VMEM((AK, AEB, D), _f32),
    pltpu.VMEM((ZR, D), _f32),
    pltpu.VMEM_SHARED((NPAD, D), _f32),
] + [pltpu.SemaphoreType.DMA] * (3 * AK)


@functools.partial(
    pl.kernel,
    out_type=(jax.ShapeDtypeStruct((3, NPAD, D), _f32),
              jax.ShapeDtypeStruct((3, NPAD, D), _f32)),
    mesh=_sc_mesh,
    scratch_types=_agg_scratch,
)
def _agg_kernel(hs0, hs1, hs2, ed_hbm, zin_hbm, p0_hbm, p1_hbm,
                idx_v, rows_v, z_v, acc_sh, *sems):
    c = lax.axis_index("c")
    s = lax.axis_index("s")
    wid = c * NS + s
    hss = [hs0, hs1, hs2]
    l_sems = sems[:AK]
    g_sems = sems[AK:2 * AK]
    s_sems = sems[2 * AK:]
    r0 = s * RPT

    @pl.loop(0, ZR)
    def _zinit(i):
        for j in range(D // 16):
            z_v[i, pl.ds(j * 16, 16)] = jnp.zeros((16,), _f32)

    for r in range(3):
        hs = hss[r]
        base = r * (E // AEB) + wid * AENB

        def _fire_load(k, i):
            pltpu.async_copy(ed_hbm.at[base + i], idx_v.at[k], l_sems[k])

        def _wait_load(k):
            pltpu.make_async_copy(ed_hbm.at[base], idx_v.at[k],
                                  l_sems[k]).wait()

        def _fire_gather(k):
            pltpu.async_copy(hs.at[idx_v.at[k, 0]], rows_v.at[k],
                             g_sems[k])

        def _wait_gather(k):
            pltpu.make_async_copy(hs.at[idx_v.at[k, 0]], rows_v.at[k],
                                  g_sems[k]).wait()

        def _fire_scatter(k):
            pltpu.make_async_copy(rows_v.at[k], acc_sh.at[idx_v.at[k, 1]],
                                  s_sems[k]).start(add=True)

        def _drain_scatter(k):
            pltpu.make_async_copy(rows_v.at[k], acc_sh.at[idx_v.at[k, 1]],
                                  s_sems[k]).wait()

        for k in range(AK):
            _fire_load(k, k)

        for t in range(RPT // ZR):
            pltpu.sync_copy(z_v, acc_sh.at[pl.ds(r0 + t * ZR, ZR)])
        plsc.subcore_barrier()

        @pl.loop(0, ARND - 1)
        def _round(o):
            for k in range(AK):
                _wait_load(k)
                _fire_gather(k)
            for k in range(AK):
                _wait_gather(k)
                _fire_scatter(k)
            for k in range(AK):
                _drain_scatter(k)
                _fire_load(k, (o + 1) * AK + k)

        for k in range(AK):
            _wait_load(k)
            _fire_gather(k)
        for k in range(AK):
            _wait_gather(k)
            _fire_scatter(k)
        for k in range(AK):
            _drain_scatter(k)
        plsc.subcore_barrier()

        @pl.when(c == 0)
        def _():
            pltpu.sync_copy(acc_sh.at[pl.ds(r0, RPT)],
                            p0_hbm.at[r, pl.ds(r0, RPT)])

        @pl.when(c == 1)
        def _():
            pltpu.sync_copy(acc_sh.at[pl.ds(r0, RPT)],
                            p1_hbm.at[r, pl.ds(r0, RPT)])


BN = 1000
NBLK = N // BN


def _scale_body(x_ref, w0_ref, w1_ref, w2_ref, deg_ref,
                h0_ref, h1_ref, h2_ref):
    xb = x_ref[...]
    outs = [h0_ref, h1_ref, h2_ref]
    ws = [w0_ref, w1_ref, w2_ref]
    for r in range(3):
        h = jnp.dot(xb, ws[r][...], preferred_element_type=_f32)
        nrm = lax.rsqrt(jnp.maximum(deg_ref[r, 0, 0, :], 1.0))
        outs[r][...] = h * nrm[:, None]


_scale_call = pl.pallas_call(
    _scale_body,
    grid=(NBLK,),
    in_specs=[
        pl.BlockSpec((BN, D), lambda b: (b, 0)),
        pl.BlockSpec((D, D), lambda b: (0, 0)),
        pl.BlockSpec((D, D), lambda b: (0, 0)),
        pl.BlockSpec((D, D), lambda b: (0, 0)),
        pl.BlockSpec((3, 1, 1, BN), lambda b: (0, b, 0, 0)),
    ],
    out_specs=[pl.BlockSpec((BN, D), lambda b: (b, 0))] * 3,
    out_shape=[jax.ShapeDtypeStruct((N, D), _f32)] * 3,
)


def _combine_body(p0_ref, p1_ref, deg_ref, out_ref):
    acc = jnp.zeros((BN, D), _f32)
    for r in range(3):
        nrm = lax.rsqrt(jnp.maximum(deg_ref[r, 0, 0, :], 1.0))
        acc = acc + (p0_ref[r] + p1_ref[r]) * nrm[:, None]
    out_ref[...] = acc


_combine_call = pl.pallas_call(
    _combine_body,
    grid=(NBLK,),
    in_specs=[
        pl.BlockSpec((3, BN, D), lambda b: (0, b, 0)),
        pl.BlockSpec((3, BN, D), lambda b: (0, b, 0)),
        pl.BlockSpec((3, 1, 1, BN), lambda b: (0, b, 0, 0)),
    ],
    out_specs=pl.BlockSpec((BN, D), lambda b: (b, 0)),
    out_shape=jax.ShapeDtypeStruct((N, D), _f32),
)


def kernel(x, edge_index_rel0, edge_index_rel1, edge_index_rel2, W0, W1, W2):
    src = jnp.stack([edge_index_rel0[0], edge_index_rel1[0],
                     edge_index_rel2[0]])
    dst = jnp.stack([edge_index_rel0[1], edge_index_rel1[1],
                     edge_index_rel2[1]])
    ed = jnp.stack([src.reshape(3, E // AEB, AEB),
                    dst.reshape(3, E // AEB, AEB)],
                   axis=2).reshape(3 * (E // AEB), 2, AEB)

    ones_in = jnp.ones((EB,), _f32)
    zeros_in = jnp.zeros((RPT,), _f32)
    deg = _deg_kernel(src[0], src[1], src[2],
                      dst[0], dst[1], dst[2],
                      ones_in, zeros_in).reshape(6, NPAD)
    deg_out = deg[:3, :N].reshape(3, NBLK, 1, BN)
    deg_in = deg[3:, :N].reshape(3, NBLK, 1, BN)

    h0, h1, h2 = _scale_call(x, W0, W1, W2, deg_out)
    zin = jnp.zeros((RPT, D), _f32)
    p0, p1 = _agg_kernel(h0, h1, h2, ed, zin)
    out = _combine_call(p0, p1, deg_in)
    return out


# speedup vs baseline: 1.0599x; 1.0181x over previous
import functools

import jax
import jax.numpy as jnp
from jax import lax
from jax.experimental import pallas as pl
from jax.experimental.pallas import tpu as pltpu
from jax.experimental.pallas import tpu_sc as plsc

N = 10000
E = 320000
D = 128

NC = 2
NS = 16
NW = NC * NS

NPAD = 10240
RPT = NPAD // NS

EB = 80
EPT = E // NW
ENB = EPT // EB

HPT = E // NS
HNB = HPT // EB

_f32 = jnp.float32
_i32 = jnp.int32

_sc_mesh = plsc.VectorSubcoreMesh(
    core_axis_name="c", subcore_axis_name="s", num_cores=NC, num_subcores=NS)


HK = 10
HRND = HNB // HK


@functools.partial(
    pl.kernel,
    out_type=jax.ShapeDtypeStruct((6 * NPAD,), _f32),
    mesh=_sc_mesh,
    scratch_types=[
        pltpu.VMEM((HK, EB), _i32),
        pltpu.VMEM((EB,), _f32),
        pltpu.VMEM((RPT,), _f32),
        pltpu.VMEM_SHARED((NPAD,), _f32),
        pltpu.VMEM_SHARED((NPAD,), _f32),
        pltpu.VMEM_SHARED((NPAD,), _f32),
    ] + [pltpu.SemaphoreType.DMA] * (2 * HK),
)
def _deg_kernel(s0, s1, s2, d0, d1, d2, ones_hbm, zeros_hbm, deg_hbm,
                idx_v, ones_v, z_v, h0, h1, h2, *sems):
    c = lax.axis_index("c")
    s = lax.axis_index("s")
    hists = [h0, h1, h2]
    srcs = [s0, s1, s2]
    dsts = [d0, d1, d2]
    ld_sems = sems[:HK]
    sc_sems = sems[HK:]

    pltpu.sync_copy(ones_hbm, ones_v)
    pltpu.sync_copy(zeros_hbm, z_v)
    for h in range(3):
        pltpu.sync_copy(z_v, hists[h].at[pl.ds(s * RPT, RPT)])
    plsc.subcore_barrier()

    for h in range(3):
        def _fire_load(k, i):
            off = s * HPT + i * EB

            @pl.when(c == 0)
            def _():
                pltpu.async_copy(srcs[h].at[pl.ds(off, EB)],
                                 idx_v.at[k], ld_sems[k])

            @pl.when(c == 1)
            def _():
                pltpu.async_copy(dsts[h].at[pl.ds(off, EB)],
                                 idx_v.at[k], ld_sems[k])

        def _wait_load(k):
            pltpu.make_async_copy(srcs[h].at[pl.ds(0, EB)],
                                  idx_v.at[k], ld_sems[k]).wait()

        def _fire_scatter(k):
            pltpu.make_async_copy(
                ones_v, hists[h].at[idx_v.at[k]],
                sc_sems[k]).start(add=True)

        def _drain_scatter(k):
            pltpu.make_async_copy(
                ones_v, hists[h].at[idx_v.at[k]], sc_sems[k]).wait()

        for k in range(HK):
            _fire_load(k, k)

        @pl.loop(0, HRND - 1)
        def _round(o):
            for k in range(HK):
                _wait_load(k)
                _fire_scatter(k)
            for k in range(HK):
                _drain_scatter(k)
                _fire_load(k, (o + 1) * HK + k)

        for k in range(HK):
            _wait_load(k)
            _fire_scatter(k)
        for k in range(HK):
            _drain_scatter(k)

    plsc.subcore_barrier()
    for h in range(3):
        @pl.when(c == 0)
        def _():
            pltpu.sync_copy(hists[h].at[pl.ds(s * RPT, RPT)],
                            deg_hbm.at[pl.ds(h * NPAD + s * RPT, RPT)])

        @pl.when(c == 1)
        def _():
            pltpu.sync_copy(hists[h].at[pl.ds(s * RPT, RPT)],
                            deg_hbm.at[pl.ds((h + 3) * NPAD + s * RPT, RPT)])


AK = 5
AEB = 50
AENB = EPT // AEB
ARND = AENB // AK
ZR = 32

_agg_scratch = [
    pltpu.VMEM((AK, 2, AEB), _i32),
    pltpu.VMEM((AK, AEB, D), _f32),
    pltpu.VMEM((ZR, D), _f32),
    pltpu.VMEM_SHARED((NPAD, D), _f32),
] + [pltpu.SemaphoreType.DMA] * (3 * AK)


@functools.partial(
    pl.kernel,
    out_type=(jax.ShapeDtypeStruct((3, NPAD, D), _f32),
              jax.ShapeDtypeStruct((3, NPAD, D), _f32)),
    mesh=_sc_mesh,
    scratch_types=_agg_scratch,
)
def _agg_kernel(hs0, hs1, hs2, ed_hbm, zin_hbm, p0_hbm, p1_hbm,
                idx_v, rows_v, z_v, acc_sh, *sems):
    c = lax.axis_index("c")
    s = lax.axis_index("s")
    wid = c * NS + s
    hss = [hs0, hs1, hs2]
    l_sems = sems[:AK]
    g_sems = sems[AK:2 * AK]
    s_sems = sems[2 * AK:]
    r0 = s * RPT

    @pl.loop(0, ZR)
    def _zinit(i):
        for j in range(D // 16):
            z_v[i, pl.ds(j * 16, 16)] = jnp.zeros((16,), _f32)

    for r in range(3):
        hs = hss[r]
        base = r * (E // AEB) + wid * AENB

        def _fire_load(k, i):
            pltpu.async_copy(ed_hbm.at[base + i], idx_v.at[k], l_sems[k])

        def _wait_load(k):
            pltpu.make_async_copy(ed_hbm.at[base], idx_v.at[k],
                                  l_sems[k]).wait()

        def _fire_gather(k):
            pltpu.async_copy(hs.at[idx_v.at[k, 0]], rows_v.at[k],
                             g_sems[k])

        def _wait_gather(k):
            pltpu.make_async_copy(hs.at[idx_v.at[k, 0]], rows_v.at[k],
                                  g_sems[k]).wait()

        def _fire_scatter(k):
            pltpu.make_async_copy(rows_v.at[k], acc_sh.at[idx_v.at[k, 1]],
                                  s_sems[k]).start(add=True)

        def _drain_scatter(k):
            pltpu.make_async_copy(rows_v.at[k], acc_sh.at[idx_v.at[k, 1]],
                                  s_sems[k]).wait()

        for k in range(AK):
            _fire_load(k, k)

        for t in range(RPT // ZR):
            pltpu.sync_copy(z_v, acc_sh.at[pl.ds(r0 + t * ZR, ZR)])
        plsc.subcore_barrier()

        @pl.loop(0, ARND - 1)
        def _round(o):
            for k in range(AK):
                _wait_load(k)
                _fire_gather(k)
            for k in range(AK):
                _wait_gather(k)
                _fire_scatter(k)
            for k in range(AK):
                _drain_scatter(k)
                _fire_load(k, (o + 1) * AK + k)

        for k in range(AK):
            _wait_load(k)
            _fire_gather(k)
        for k in range(AK):
            _wait_gather(k)
            _fire_scatter(k)
        for k in range(AK):
            _drain_scatter(k)
        plsc.subcore_barrier()

        @pl.when(c == 0)
        def _():
            pltpu.sync_copy(acc_sh.at[pl.ds(r0, RPT)],
                            p0_hbm.at[r, pl.ds(r0, RPT)])

        @pl.when(c == 1)
        def _():
            pltpu.sync_copy(acc_sh.at[pl.ds(r0, RPT)],
                            p1_hbm.at[r, pl.ds(r0, RPT)])


BN = 1000
NBLK = N // BN


def _scale_body(x_ref, w0_ref, w1_ref, w2_ref, deg_ref,
                h0_ref, h1_ref, h2_ref):
    xb = x_ref[...]
    outs = [h0_ref, h1_ref, h2_ref]
    ws = [w0_ref, w1_ref, w2_ref]
    for r in range(3):
        h = jnp.dot(xb, ws[r][...], preferred_element_type=_f32)
        nrm = lax.rsqrt(jnp.maximum(deg_ref[r, 0, 0, :], 1.0))
        outs[r][...] = h * nrm[:, None]


_scale_call = pl.pallas_call(
    _scale_body,
    grid=(NBLK,),
    in_specs=[
        pl.BlockSpec((BN, D), lambda b: (b, 0)),
        pl.BlockSpec((D, D), lambda b: (0, 0)),
        pl.BlockSpec((D, D), lambda b: (0, 0)),
        pl.BlockSpec((D, D), lambda b: (0, 0)),
        pl.BlockSpec((3, 1, 1, BN), lambda b: (0, b, 0, 0)),
    ],
    out_specs=[pl.BlockSpec((BN, D), lambda b: (b, 0))] * 3,
    out_shape=[jax.ShapeDtypeStruct((N, D), _f32)] * 3,
)


def _combine_body(p0_ref, p1_ref, deg_ref, out_ref):
    acc = jnp.zeros((BN, D), _f32)
    for r in range(3):
        nrm = lax.rsqrt(jnp.maximum(deg_ref[r, 0, 0, :], 1.0))
        acc = acc + (p0_ref[r] + p1_ref[r]) * nrm[:, None]
    out_ref[...] = acc


_combine_call = pl.pallas_call(
    _combine_body,
    grid=(NBLK,),
    in_specs=[
        pl.BlockSpec((3, BN, D), lambda b: (0, b, 0)),
        pl.BlockSpec((3, BN, D), lambda b: (0, b, 0)),
        pl.BlockSpec((3, 1, 1, BN), lambda b: (0, b, 0, 0)),
    ],
    out_specs=pl.BlockSpec((BN, D), lambda b: (b, 0)),
    out_shape=jax.ShapeDtypeStruct((N, D), _f32),
)


def kernel(x, edge_index_rel0, edge_index_rel1, edge_index_rel2, W0, W1, W2):
    src = jnp.stack([edge_index_rel0[0], edge_index_rel1[0],
                     edge_index_rel2[0]])
    dst = jnp.stack([edge_index_rel0[1], edge_index_rel1[1],
                     edge_index_rel2[1]])
    ed = jnp.stack([src.reshape(3, E // AEB, AEB),
                    dst.reshape(3, E // AEB, AEB)],
                   axis=2).reshape(3 * (E // AEB), 2, AEB)

    ones_in = jnp.ones((EB,), _f32)
    zeros_in = jnp.zeros((RPT,), _f32)
    deg = _deg_kernel(src[0], src[1], src[2],
                      dst[0], dst[1], dst[2],
                      ones_in, zeros_in).reshape(6, NPAD)
    deg_out = deg[:3, :N].reshape(3, NBLK, 1, BN)
    deg_in = deg[3:, :N].reshape(3, NBLK, 1, BN)

    h0, h1, h2 = _scale_call(x, W0, W1, W2, deg_out)
    zin = jnp.zeros((RPT, D), _f32)
    p0, p1 = _agg_kernel(h0, h1, h2, ed, zin)
    out = _combine_call(p0, p1, deg_in)
    return out
